# baseline (speedup 1.0000x reference)
import functools

import jax
import jax.numpy as jnp
from jax import lax
from jax.experimental import pallas as pl
from jax.experimental.pallas import tpu as pltpu
from jax.experimental.pallas import tpu_sc as plsc

N = 10000
E = 320000
F_IN = 128
D_IN = 64
H = 128

NC = 2
NS = 16
NW = NC * NS
C = 80
EPT = E // NW
NCH = EPT // C
NPAD = 10240
RPT = NPAD // NS
C2 = 128
NCHG = E // C2
NCHP = NCHG + 4
NCH2 = 80
HNCH = NCH2 // 2

_MESH = plsc.VectorSubcoreMesh(
    core_axis_name="c", subcore_axis_name="s", num_cores=NC, num_subcores=NS
)


@functools.partial(
    pl.kernel,
    out_type=jax.ShapeDtypeStruct((NC, 2, NPAD), jnp.float32),
    mesh=_MESH,
    scratch_types=[
        pltpu.VMEM((NCH2, C2), jnp.int32),
        pltpu.VMEM((NCH2, C2), jnp.int32),
        pltpu.VMEM((C2,), jnp.float32),
        pltpu.VMEM_SHARED((NPAD,), jnp.float32),
        pltpu.VMEM_SHARED((NPAD,), jnp.float32),
        pltpu.SemaphoreType.DMA,
    ],
)
def _sc_degrees(src_hbm, dst_hbm, zvec_hbm, out_hbm, sidx, didx, ones, acc_in, acc_out, sem):
    c = lax.axis_index("c")
    s = lax.axis_index("s")
    w = c * NS + s
    start = w * NCH2
    nch = jnp.minimum(NCHG - start, NCH2)
    off = pl.multiple_of(jnp.minimum(start, NCHP - NCH2), 8)
    jb = start - off
    cps = [
        pltpu.async_copy(src_hbm.at[pl.ds(off, NCH2)], sidx, sem),
        pltpu.async_copy(dst_hbm.at[pl.ds(off, NCH2)], didx, sem),
        pltpu.async_copy(zvec_hbm, acc_in.at[pl.ds(s * RPT, RPT)], sem),
        pltpu.async_copy(zvec_hbm, acc_out.at[pl.ds(s * RPT, RPT)], sem),
    ]
    for i in range(C2 // 16):
        ones[pl.ds(i * 16, 16)] = jnp.ones((16,), jnp.float32)
    for cp in cps:
        cp.wait()
    plsc.subcore_barrier()

    def body(j, carry):
        pltpu.async_copy(ones, acc_in.at[didx.at[j]], sem, add=True)
        pltpu.async_copy(ones, acc_out.at[sidx.at[j]], sem, add=True)
        return carry

    def drain(j, carry):
        pltpu.make_async_copy(ones, acc_in.at[didx.at[j]], sem).wait()
        pltpu.make_async_copy(ones, acc_out.at[sidx.at[j]], sem).wait()
        return carry

    lax.fori_loop(jb, jb + nch, body, 0)
    lax.fori_loop(jb, jb + nch, drain, 0)
    plsc.subcore_barrier()
    pltpu.sync_copy(acc_in.at[pl.ds(s * RPT, RPT)], out_hbm.at[c, 0, pl.ds(s * RPT, RPT)])
    pltpu.sync_copy(acc_out.at[pl.ds(s * RPT, RPT)], out_hbm.at[c, 1, pl.ds(s * RPT, RPT)])


@functools.partial(
    pl.kernel,
    out_type=jax.ShapeDtypeStruct((NC, NPAD, H), jnp.float32),
    mesh=_MESH,
    scratch_types=[
        pltpu.VMEM((HNCH, C2), jnp.int32),
        pltpu.VMEM((HNCH, C2), jnp.int32),
        pltpu.VMEM((C2, H), jnp.float32),
        pltpu.VMEM((C2, H), jnp.float32),
        pltpu.VMEM_SHARED((NPAD, H), jnp.float32),
        pltpu.SemaphoreType.DMA,
        pltpu.SemaphoreType.DMA,
        pltpu.SemaphoreType.DMA,
        pltpu.SemaphoreType.DMA,
    ],
)
def _sc_conv(xn_hbm, src_hbm, dst_hbm, zrows_hbm, out_hbm, sidx, didx,
             rows0, rows1, acc, gsem0, gsem1, ssem0, ssem1):
    c = lax.axis_index("c")
    s = lax.axis_index("s")
    w = c * NS + s
    zcp = pltpu.async_copy(zrows_hbm, acc.at[pl.ds(s * RPT, RPT)], gsem1)
    nch = jnp.minimum(NCHG - w * NCH2, NCH2)

    for half in range(2):
        start = w * NCH2 + half * HNCH
        cnt = jnp.clip(nch - half * HNCH, 0, HNCH)
        off = pl.multiple_of(jnp.minimum(start, NCHP - HNCH), 8)
        jb = start - off
        npairs = cnt // 2

        scp = pltpu.async_copy(src_hbm.at[pl.ds(off, HNCH)], sidx, gsem0)
        dcp = pltpu.async_copy(dst_hbm.at[pl.ds(off, HNCH)], didx, gsem1)
        scp.wait()
        dcp.wait()
        if half == 0:
            zcp.wait()
            plsc.subcore_barrier()

        @pl.when(npairs > 0)
        def _half():
            pltpu.async_copy(xn_hbm.at[sidx.at[jb]], rows0, gsem0)
            pltpu.async_copy(xn_hbm.at[sidx.at[jb + 1]], rows1, gsem1)

            def body(k, carry):
                j0 = jb + 2 * k
                pltpu.make_async_copy(xn_hbm.at[sidx.at[j0]], rows0, gsem0).wait()
                pltpu.async_copy(rows0, acc.at[didx.at[j0]], ssem0, add=True)
                pltpu.make_async_copy(xn_hbm.at[sidx.at[j0 + 1]], rows1, gsem1).wait()
                pltpu.async_copy(rows1, acc.at[didx.at[j0 + 1]], ssem1, add=True)

                @pl.when(k < npairs - 1)
                def _():
                    pltpu.make_async_copy(rows0, acc.at[didx.at[j0]], ssem0).wait()
                    pltpu.async_copy(xn_hbm.at[sidx.at[j0 + 2]], rows0, gsem0)
                    pltpu.make_async_copy(rows1, acc.at[didx.at[j0 + 1]], ssem1).wait()
                    pltpu.async_copy(xn_hbm.at[sidx.at[j0 + 3]], rows1, gsem1)

                return carry

            lax.fori_loop(0, npairs, body, 0)
            pltpu.make_async_copy(rows0, acc.at[didx.at[jb]], ssem0).wait()
            pltpu.make_async_copy(rows1, acc.at[didx.at[jb]], ssem1).wait()

    plsc.subcore_barrier()
    pltpu.sync_copy(acc.at[pl.ds(s * RPT, RPT)], out_hbm.at[c, pl.ds(s * RPT, RPT)])


_R = 400
_PREC = lax.Precision.DEFAULT


def _norms_from_deg(d):
    deg_in = d[:, 0:1] + d[:, 2:3]
    deg_out = d[:, 1:2] + d[:, 3:4]
    norm_dst = lax.rsqrt(jnp.maximum(deg_in, 1.0))
    norm_src = lax.rsqrt(jnp.maximum(deg_out, 1.0))
    return norm_src, norm_dst


def _project_body(f_ref, nt_ref, deg_ref, wp_ref, bp_ref, wd_ref, bd_ref, out_ref):
    f = f_ref[...]
    hp = jnp.dot(f, wp_ref[...], preferred_element_type=jnp.float32, precision=_PREC)
    hd = jnp.dot(f[:, :D_IN], wd_ref[...], preferred_element_type=jnp.float32, precision=_PREC)
    h = jnp.where(nt_ref[...] == 0, hp + bp_ref[...], hd + bd_ref[...])
    norm_src, _ = _norms_from_deg(deg_ref[...])
    out_ref[...] = h * norm_src


def _tc_project(features, nt2, degT, Wp, bp, Wd, bd):
    grid = (N // _R,)
    return pl.pallas_call(
        _project_body,
        grid=grid,
        in_specs=[
            pl.BlockSpec((_R, F_IN), lambda i: (i, 0)),
            pl.BlockSpec((_R, 1), lambda i: (i, 0)),
            pl.BlockSpec((_R, 4), lambda i: (i, 0)),
            pl.BlockSpec((F_IN, H), lambda i: (0, 0)),
            pl.BlockSpec((1, H), lambda i: (0, 0)),
            pl.BlockSpec((D_IN, H), lambda i: (0, 0)),
            pl.BlockSpec((1, H), lambda i: (0, 0)),
        ],
        out_specs=pl.BlockSpec((_R, H), lambda i: (i, 0)),
        out_shape=jax.ShapeDtypeStruct((N, H), jnp.float32),
    )(features, nt2, degT, Wp, bp, Wd, bd)


def _make_post_body(relu, scale_src):
    def body(agg_ref, deg_ref, w_ref, b_ref, out_ref):
        a = agg_ref[...]
        norm_src, norm_dst = _norms_from_deg(deg_ref[...])
        agg = (a[0] + a[1]) * norm_dst
        y = jnp.dot(agg, w_ref[...], preferred_element_type=jnp.float32, precision=_PREC)
        y = y + b_ref[...]
        if relu:
            y = jnp.maximum(y, 0.0)
        if scale_src:
            y = y * norm_src
        out_ref[...] = y
    return body


def _tc_post(aggp, degT, W, b, relu, scale_src):
    grid = (N // _R,)
    return pl.pallas_call(
        _make_post_body(relu, scale_src),
        grid=grid,
        in_specs=[
            pl.BlockSpec((NC, _R, H), lambda i: (0, i, 0)),
            pl.BlockSpec((_R, 4), lambda i: (i, 0)),
            pl.BlockSpec((H, H), lambda i: (0, 0)),
            pl.BlockSpec((1, H), lambda i: (0, 0)),
        ],
        out_specs=pl.BlockSpec((_R, H), lambda i: (i, 0)),
        out_shape=jax.ShapeDtypeStruct((N, H), jnp.float32),
    )(aggp, degT, W, b)


def kernel(features, edge_index, node_type, W_person, b_person, W_disease,
           b_disease, W1, b1, W2, b2):
    epad = jnp.zeros(((NCHP - NCHG) * C2,), jnp.int32)
    srcp = jnp.concatenate([edge_index[0], epad]).reshape(NCHP, C2)
    dstp = jnp.concatenate([edge_index[1], epad]).reshape(NCHP, C2)
    zvec = jnp.zeros((RPT,), jnp.float32)
    zrows = jnp.zeros((RPT, H), jnp.float32)

    degp = _sc_degrees(srcp, dstp, zvec)
    degT = jnp.moveaxis(degp[:, :, :N], 2, 0).reshape(N, NC * 2)

    nt2 = node_type.reshape(N, 1)
    xn1 = _tc_project(features, nt2, degT, W_person, b_person.reshape(1, H),
                      W_disease, b_disease.reshape(1, H))

    aggp1 = _sc_conv(xn1, srcp, dstp, zrows)
    xn2 = _tc_post(aggp1, degT, W1, b1.reshape(1, H), relu=True, scale_src=True)

    aggp2 = _sc_conv(xn2, srcp, dstp, zrows)
    z = _tc_post(aggp2, degT, W2, b2.reshape(1, H), relu=False, scale_src=False)
    return z


# speedup vs baseline: 1.2170x; 1.2170x over previous
import functools

import jax
import jax.numpy as jnp
from jax import lax
from jax.experimental import pallas as pl
from jax.experimental.pallas import tpu as pltpu
from jax.experimental.pallas import tpu_sc as plsc

N = 10000
E = 320000
F_IN = 128
D_IN = 64
H = 128

NC = 2
NS = 16
NW = NC * NS
C = 80
EPT = E // NW
NCH = EPT // C
NPAD = 10240
RPT = NPAD // NS
C2 = 128
NCHG = E // C2
NCHP = NCHG + 4
NCH2 = 80
HNCH = NCH2 // 2

_MESH = plsc.VectorSubcoreMesh(
    core_axis_name="c", subcore_axis_name="s", num_cores=NC, num_subcores=NS
)


@functools.partial(
    pl.kernel,
    out_type=jax.ShapeDtypeStruct((NC, 2, NPAD), jnp.float32),
    mesh=_MESH,
    scratch_types=[
        pltpu.VMEM((NCH2, C2), jnp.int32),
        pltpu.VMEM((NCH2, C2), jnp.int32),
        pltpu.VMEM((C2,), jnp.float32),
        pltpu.VMEM_SHARED((NPAD,), jnp.float32),
        pltpu.VMEM_SHARED((NPAD,), jnp.float32),
        pltpu.SemaphoreType.DMA,
    ],
)
def _sc_degrees(src_hbm, dst_hbm, zvec_hbm, out_hbm, sidx, didx, ones, acc_in, acc_out, sem):
    c = lax.axis_index("c")
    s = lax.axis_index("s")
    w = c * NS + s
    start = w * NCH2
    nch = jnp.minimum(NCHG - start, NCH2)
    off = pl.multiple_of(jnp.minimum(start, NCHP - NCH2), 8)
    jb = start - off
    cps = [
        pltpu.async_copy(src_hbm.at[pl.ds(off, NCH2)], sidx, sem),
        pltpu.async_copy(dst_hbm.at[pl.ds(off, NCH2)], didx, sem),
        pltpu.async_copy(zvec_hbm, acc_in.at[pl.ds(s * RPT, RPT)], sem),
        pltpu.async_copy(zvec_hbm, acc_out.at[pl.ds(s * RPT, RPT)], sem),
    ]
    for i in range(C2 // 16):
        ones[pl.ds(i * 16, 16)] = jnp.ones((16,), jnp.float32)
    for cp in cps:
        cp.wait()
    plsc.subcore_barrier()

    def body(j, carry):
        pltpu.async_copy(ones, acc_in.at[didx.at[j]], sem, add=True)
        pltpu.async_copy(ones, acc_out.at[sidx.at[j]], sem, add=True)
        return carry

    def drain(j, carry):
        pltpu.make_async_copy(ones, acc_in.at[didx.at[j]], sem).wait()
        pltpu.make_async_copy(ones, acc_out.at[sidx.at[j]], sem).wait()
        return carry

    lax.fori_loop(jb, jb + nch, body, 0)
    lax.fori_loop(jb, jb + nch, drain, 0)
    plsc.subcore_barrier()
    pltpu.sync_copy(acc_in.at[pl.ds(s * RPT, RPT)], out_hbm.at[c, 0, pl.ds(s * RPT, RPT)])
    pltpu.sync_copy(acc_out.at[pl.ds(s * RPT, RPT)], out_hbm.at[c, 1, pl.ds(s * RPT, RPT)])


@functools.partial(
    pl.kernel,
    out_type=jax.ShapeDtypeStruct((NC, NPAD, H), jnp.float32),
    mesh=_MESH,
    scratch_types=[
        pltpu.VMEM((HNCH, C2), jnp.int32),
        pltpu.VMEM((HNCH, C2), jnp.int32),
        pltpu.VMEM((C2, H), jnp.float32),
        pltpu.VMEM((C2, H), jnp.float32),
        pltpu.VMEM_SHARED((NPAD, H), jnp.float32),
        pltpu.SemaphoreType.DMA,
        pltpu.SemaphoreType.DMA,
    ],
)
def _sc_conv(xn_hbm, src_hbm, dst_hbm, zrows_hbm, out_hbm, sidx, didx,
             rows0, rows1, acc, gsem0, gsem1):
    c = lax.axis_index("c")
    s = lax.axis_index("s")
    w = c * NS + s
    zcp = pltpu.async_copy(zrows_hbm, acc.at[pl.ds(s * RPT, RPT)], gsem1)
    nch = jnp.minimum(NCHG - w * NCH2, NCH2)

    for half in range(2):
        start = w * NCH2 + half * HNCH
        cnt = jnp.clip(nch - half * HNCH, 0, HNCH)
        off = pl.multiple_of(jnp.minimum(start, NCHP - HNCH), 8)
        jb = start - off
        npairs = cnt // 2

        scp = pltpu.async_copy(src_hbm.at[pl.ds(off, HNCH)], sidx, gsem0)
        dcp = pltpu.async_copy(dst_hbm.at[pl.ds(off, HNCH)], didx, gsem1)
        scp.wait()
        dcp.wait()
        if half == 0:
            zcp.wait()
            plsc.subcore_barrier()

        @pl.when(npairs > 0)
        def _half():
            pltpu.async_copy(xn_hbm.at[sidx.at[jb]], rows0, gsem0)

            def body(k, carry):
                j0 = jb + 2 * k
                pltpu.async_copy(xn_hbm.at[sidx.at[j0 + 1]], rows1, gsem1)
                pltpu.make_async_copy(xn_hbm.at[sidx.at[j0]], rows0, gsem0).wait()
                pltpu.sync_copy(rows0, acc.at[didx.at[j0]], add=True)

                @pl.when(k < npairs - 1)
                def _():
                    pltpu.async_copy(xn_hbm.at[sidx.at[j0 + 2]], rows0, gsem0)

                pltpu.make_async_copy(xn_hbm.at[sidx.at[j0 + 1]], rows1, gsem1).wait()
                pltpu.sync_copy(rows1, acc.at[didx.at[j0 + 1]], add=True)
                return carry

            lax.fori_loop(0, npairs, body, 0)

    plsc.subcore_barrier()
    pltpu.sync_copy(acc.at[pl.ds(s * RPT, RPT)], out_hbm.at[c, pl.ds(s * RPT, RPT)])


_R = 400
_PREC = lax.Precision.DEFAULT


def _norms_from_deg(d):
    deg_in = d[:, 0:1] + d[:, 2:3]
    deg_out = d[:, 1:2] + d[:, 3:4]
    norm_dst = lax.rsqrt(jnp.maximum(deg_in, 1.0))
    norm_src = lax.rsqrt(jnp.maximum(deg_out, 1.0))
    return norm_src, norm_dst


def _project_body(f_ref, nt_ref, deg_ref, wp_ref, bp_ref, wd_ref, bd_ref, out_ref):
    f = f_ref[...]
    hp = jnp.dot(f, wp_ref[...], preferred_element_type=jnp.float32, precision=_PREC)
    hd = jnp.dot(f[:, :D_IN], wd_ref[...], preferred_element_type=jnp.float32, precision=_PREC)
    h = jnp.where(nt_ref[...] == 0, hp + bp_ref[...], hd + bd_ref[...])
    norm_src, _ = _norms_from_deg(deg_ref[...])
    out_ref[...] = h * norm_src


def _tc_project(features, nt2, degT, Wp, bp, Wd, bd):
    grid = (N // _R,)
    return pl.pallas_call(
        _project_body,
        grid=grid,
        in_specs=[
            pl.BlockSpec((_R, F_IN), lambda i: (i, 0)),
            pl.BlockSpec((_R, 1), lambda i: (i, 0)),
            pl.BlockSpec((_R, 4), lambda i: (i, 0)),
            pl.BlockSpec((F_IN, H), lambda i: (0, 0)),
            pl.BlockSpec((1, H), lambda i: (0, 0)),
            pl.BlockSpec((D_IN, H), lambda i: (0, 0)),
            pl.BlockSpec((1, H), lambda i: (0, 0)),
        ],
        out_specs=pl.BlockSpec((_R, H), lambda i: (i, 0)),
        out_shape=jax.ShapeDtypeStruct((N, H), jnp.float32),
    )(features, nt2, degT, Wp, bp, Wd, bd)


def _make_post_body(relu, scale_src):
    def body(agg_ref, deg_ref, w_ref, b_ref, out_ref):
        a = agg_ref[...]
        norm_src, norm_dst = _norms_from_deg(deg_ref[...])
        agg = (a[0] + a[1]) * norm_dst
        y = jnp.dot(agg, w_ref[...], preferred_element_type=jnp.float32, precision=_PREC)
        y = y + b_ref[...]
        if relu:
            y = jnp.maximum(y, 0.0)
        if scale_src:
            y = y * norm_src
        out_ref[...] = y
    return body


def _tc_post(aggp, degT, W, b, relu, scale_src):
    grid = (N // _R,)
    return pl.pallas_call(
        _make_post_body(relu, scale_src),
        grid=grid,
        in_specs=[
            pl.BlockSpec((NC, _R, H), lambda i: (0, i, 0)),
            pl.BlockSpec((_R, 4), lambda i: (i, 0)),
            pl.BlockSpec((H, H), lambda i: (0, 0)),
            pl.BlockSpec((1, H), lambda i: (0, 0)),
        ],
        out_specs=pl.BlockSpec((_R, H), lambda i: (i, 0)),
        out_shape=jax.ShapeDtypeStruct((N, H), jnp.float32),
    )(aggp, degT, W, b)


def kernel(features, edge_index, node_type, W_person, b_person, W_disease,
           b_disease, W1, b1, W2, b2):
    epad = jnp.zeros(((NCHP - NCHG) * C2,), jnp.int32)
    srcp = jnp.concatenate([edge_index[0], epad]).reshape(NCHP, C2)
    dstp = jnp.concatenate([edge_index[1], epad]).reshape(NCHP, C2)
    zvec = jnp.zeros((RPT,), jnp.float32)
    zrows = jnp.zeros((RPT, H), jnp.float32)

    degp = _sc_degrees(srcp, dstp, zvec)
    degT = jnp.moveaxis(degp[:, :, :N], 2, 0).reshape(N, NC * 2)

    nt2 = node_type.reshape(N, 1)
    xn1 = _tc_project(features, nt2, degT, W_person, b_person.reshape(1, H),
                      W_disease, b_disease.reshape(1, H))

    aggp1 = _sc_conv(xn1, srcp, dstp, zrows)
    xn2 = _tc_post(aggp1, degT, W1, b1.reshape(1, H), relu=True, scale_src=True)

    aggp2 = _sc_conv(xn2, srcp, dstp, zrows)
    z = _tc_post(aggp2, degT, W2, b2.reshape(1, H), relu=False, scale_src=False)
    return z


# speedup vs baseline: 1.2214x; 1.0036x over previous
import functools

import jax
import jax.numpy as jnp
from jax import lax
from jax.experimental import pallas as pl
from jax.experimental.pallas import tpu as pltpu
from jax.experimental.pallas import tpu_sc as plsc

N = 10000
E = 320000
F_IN = 128
D_IN = 64
H = 128

NC = 2
NS = 16
NW = NC * NS
NPAD = 10240
RPT = NPAD // NS
C2 = 128
NCHG = E // C2
NCHP = NCHG + 4
NCH2 = 80
HNCH = NCH2 // 2

_MESH = plsc.VectorSubcoreMesh(
    core_axis_name="c", subcore_axis_name="s", num_cores=NC, num_subcores=NS
)


@functools.partial(
    pl.kernel,
    out_type=jax.ShapeDtypeStruct((NC, 2, NPAD), jnp.float32),
    mesh=_MESH,
    scratch_types=[
        pltpu.VMEM((NCH2, C2), jnp.int32),
        pltpu.VMEM((NCH2, C2), jnp.int32),
        pltpu.VMEM((C2,), jnp.float32),
        pltpu.VMEM_SHARED((NPAD,), jnp.float32),
        pltpu.VMEM_SHARED((NPAD,), jnp.float32),
        pltpu.SemaphoreType.DMA,
    ],
)
def _sc_degrees(src_hbm, dst_hbm, zvec_hbm, out_hbm, sidx, didx, ones, acc_in, acc_out, sem):
    c = lax.axis_index("c")
    s = lax.axis_index("s")
    w = c * NS + s
    start = w * NCH2
    nch = jnp.minimum(NCHG - start, NCH2)
    off = pl.multiple_of(jnp.minimum(start, NCHP - NCH2), 8)
    jb = start - off
    cps = [
        pltpu.async_copy(src_hbm.at[pl.ds(off, NCH2)], sidx, sem),
        pltpu.async_copy(dst_hbm.at[pl.ds(off, NCH2)], didx, sem),
        pltpu.async_copy(zvec_hbm, acc_in.at[pl.ds(s * RPT, RPT)], sem),
        pltpu.async_copy(zvec_hbm, acc_out.at[pl.ds(s * RPT, RPT)], sem),
    ]
    for i in range(C2 // 16):
        ones[pl.ds(i * 16, 16)] = jnp.ones((16,), jnp.float32)
    for cp in cps:
        cp.wait()
    plsc.subcore_barrier()

    def body(j, carry):
        pltpu.async_copy(ones, acc_in.at[didx.at[j]], sem, add=True)
        pltpu.async_copy(ones, acc_out.at[sidx.at[j]], sem, add=True)
        return carry

    def drain(j, carry):
        pltpu.make_async_copy(ones, acc_in.at[didx.at[j]], sem).wait()
        pltpu.make_async_copy(ones, acc_out.at[sidx.at[j]], sem).wait()
        return carry

    lax.fori_loop(jb, jb + nch, body, 0)
    lax.fori_loop(jb, jb + nch, drain, 0)
    plsc.subcore_barrier()
    pltpu.sync_copy(acc_in.at[pl.ds(s * RPT, RPT)], out_hbm.at[c, 0, pl.ds(s * RPT, RPT)])
    pltpu.sync_copy(acc_out.at[pl.ds(s * RPT, RPT)], out_hbm.at[c, 1, pl.ds(s * RPT, RPT)])


@functools.partial(
    pl.kernel,
    out_type=jax.ShapeDtypeStruct((NC, NPAD, H), jnp.float32),
    mesh=_MESH,
    scratch_types=[
        pltpu.VMEM((HNCH, C2), jnp.int32),
        pltpu.VMEM((HNCH, C2), jnp.int32),
        pltpu.VMEM((C2, H), jnp.float32),
        pltpu.VMEM((C2, H), jnp.float32),
        pltpu.VMEM_SHARED((NPAD, H), jnp.float32),
        pltpu.SemaphoreType.DMA,
        pltpu.SemaphoreType.DMA,
    ],
)
def _sc_conv(xn_hbm, src_hbm, dst_hbm, zrows_hbm, out_hbm, sidx, didx,
             rows0, rows1, acc, gsem0, gsem1):
    c = lax.axis_index("c")
    s = lax.axis_index("s")
    w = c * NS + s
    zcp = pltpu.async_copy(zrows_hbm, acc.at[pl.ds(s * RPT, RPT)], gsem1)
    nch = jnp.minimum(NCHG - w * NCH2, NCH2)

    for half in range(2):
        start = w * NCH2 + half * HNCH
        cnt = jnp.clip(nch - half * HNCH, 0, HNCH)
        off = pl.multiple_of(jnp.minimum(start, NCHP - HNCH), 8)
        jb = start - off
        npairs = cnt // 2

        scp = pltpu.async_copy(src_hbm.at[pl.ds(off, HNCH)], sidx, gsem0)
        dcp = pltpu.async_copy(dst_hbm.at[pl.ds(off, HNCH)], didx, gsem1)
        scp.wait()
        dcp.wait()
        if half == 0:
            zcp.wait()
            plsc.subcore_barrier()

        @pl.when(npairs > 0)
        def _half():
            pltpu.async_copy(xn_hbm.at[sidx.at[jb]], rows0, gsem0)

            def body(k, carry):
                j0 = jb + 2 * k
                pltpu.async_copy(xn_hbm.at[sidx.at[j0 + 1]], rows1, gsem1)
                pltpu.make_async_copy(xn_hbm.at[sidx.at[j0]], rows0, gsem0).wait()
                pltpu.sync_copy(rows0, acc.at[didx.at[j0]], add=True)

                @pl.when(k < npairs - 1)
                def _():
                    pltpu.async_copy(xn_hbm.at[sidx.at[j0 + 2]], rows0, gsem0)

                pltpu.make_async_copy(xn_hbm.at[sidx.at[j0 + 1]], rows1, gsem1).wait()
                pltpu.sync_copy(rows1, acc.at[didx.at[j0 + 1]], add=True)
                return carry

            lax.fori_loop(0, npairs, body, 0)

    plsc.subcore_barrier()
    pltpu.sync_copy(acc.at[pl.ds(s * RPT, RPT)], out_hbm.at[c, pl.ds(s * RPT, RPT)])


_R = 400
_PREC = lax.Precision.DEFAULT


def _norms_from_deg(d):
    deg_in = d[:, 0:1] + d[:, 2:3]
    deg_out = d[:, 1:2] + d[:, 3:4]
    norm_dst = lax.rsqrt(jnp.maximum(deg_in, 1.0))
    norm_src = lax.rsqrt(jnp.maximum(deg_out, 1.0))
    return norm_src, norm_dst


def _project_body(f_ref, nt_ref, deg_ref, wp_ref, bp_ref, wd_ref, bd_ref, out_ref):
    f = f_ref[...]
    hp = jnp.dot(f, wp_ref[...], preferred_element_type=jnp.float32, precision=_PREC)
    hd = jnp.dot(f[:, :D_IN], wd_ref[...], preferred_element_type=jnp.float32, precision=_PREC)
    h = jnp.where(nt_ref[...] == 0, hp + bp_ref[...], hd + bd_ref[...])
    norm_src, _ = _norms_from_deg(deg_ref[...])
    out_ref[...] = h * norm_src


def _tc_project(features, nt2, degT, Wp, bp, Wd, bd):
    grid = (N // _R,)
    return pl.pallas_call(
        _project_body,
        grid=grid,
        in_specs=[
            pl.BlockSpec((_R, F_IN), lambda i: (i, 0)),
            pl.BlockSpec((_R, 1), lambda i: (i, 0)),
            pl.BlockSpec((_R, 4), lambda i: (i, 0)),
            pl.BlockSpec((F_IN, H), lambda i: (0, 0)),
            pl.BlockSpec((1, H), lambda i: (0, 0)),
            pl.BlockSpec((D_IN, H), lambda i: (0, 0)),
            pl.BlockSpec((1, H), lambda i: (0, 0)),
        ],
        out_specs=pl.BlockSpec((_R, H), lambda i: (i, 0)),
        out_shape=jax.ShapeDtypeStruct((N, H), jnp.float32),
    )(features, nt2, degT, Wp, bp, Wd, bd)


def _make_post_body(relu, scale_src):
    def body(agg_ref, deg_ref, w_ref, b_ref, out_ref):
        a = agg_ref[...]
        norm_src, norm_dst = _norms_from_deg(deg_ref[...])
        agg = (a[0] + a[1]) * norm_dst
        y = jnp.dot(agg, w_ref[...], preferred_element_type=jnp.float32, precision=_PREC)
        y = y + b_ref[...]
        if relu:
            y = jnp.maximum(y, 0.0)
        if scale_src:
            y = y * norm_src
        out_ref[...] = y
    return body


def _tc_post(aggp, degT, W, b, relu, scale_src):
    grid = (N // _R,)
    return pl.pallas_call(
        _make_post_body(relu, scale_src),
        grid=grid,
        in_specs=[
            pl.BlockSpec((NC, _R, H), lambda i: (0, i, 0)),
            pl.BlockSpec((_R, 4), lambda i: (i, 0)),
            pl.BlockSpec((H, H), lambda i: (0, 0)),
            pl.BlockSpec((1, H), lambda i: (0, 0)),
        ],
        out_specs=pl.BlockSpec((_R, H), lambda i: (i, 0)),
        out_shape=jax.ShapeDtypeStruct((N, H), jnp.float32),
    )(aggp, degT, W, b)


def kernel(features, edge_index, node_type, W_person, b_person, W_disease,
           b_disease, W1, b1, W2, b2):
    epad = jnp.zeros(((NCHP - NCHG) * C2,), jnp.int32)
    srcp = jnp.concatenate([edge_index[0], epad]).reshape(NCHP, C2)
    dstp = jnp.concatenate([edge_index[1], epad]).reshape(NCHP, C2)
    zvec = jnp.zeros((RPT,), jnp.float32)
    zrows = jnp.zeros((RPT, H), jnp.float32)

    degp = _sc_degrees(srcp, dstp, zvec)
    degT = jnp.moveaxis(degp[:, :, :N], 2, 0).reshape(N, NC * 2)

    nt2 = node_type.reshape(N, 1)
    xn1 = _tc_project(features, nt2, degT, W_person, b_person.reshape(1, H),
                      W_disease, b_disease.reshape(1, H))

    aggp1 = _sc_conv(xn1, srcp, dstp, zrows)
    xn2 = _tc_post(aggp1, degT, W1, b1.reshape(1, H), relu=True, scale_src=True)

    aggp2 = _sc_conv(xn2, srcp, dstp, zrows)
    z = _tc_post(aggp2, degT, W2, b2.reshape(1, H), relu=False, scale_src=False)
    return z
